# Initial kernel scaffold; baseline (speedup 1.0000x reference)
import jax
import jax.numpy as jnp
from jax.experimental import pallas as pl


def kernel(x_categorical, weight, embedding_bias):
    raise NotImplementedError("write your pallas kernel here")


# speedup vs baseline: 3.1375x; 3.1375x over previous
import functools

import jax
import jax.numpy as jnp
from jax import lax
from jax.experimental import pallas as pl
from jax.experimental.pallas import tpu as pltpu
from jax.experimental.pallas import tpu_sc as plsc

N_FEAT = 1000
DIM = 64
BATCH = 1024
FC = 5
FCHUNK = N_FEAT // FC
CHUNK_ELEMS = FCHUNK * DIM


def kernel(x_categorical, weight, embedding_bias):
    idx = x_categorical.astype(jnp.int32).reshape(-1)
    wflat = weight.reshape(-1)
    bflat = embedding_bias.reshape(-1)

    info = plsc.get_sparse_core_info()
    nc, ns = info.num_cores, info.num_subcores
    nw = nc * ns
    bpw = BATCH // nw

    mesh = plsc.VectorSubcoreMesh(core_axis_name="c", subcore_axis_name="s")

    @functools.partial(
        pl.kernel,
        out_type=jax.ShapeDtypeStruct((BATCH * N_FEAT * DIM,), jnp.float32),
        mesh=mesh,
        compiler_params=pltpu.CompilerParams(needs_layout_passes=False),
        scratch_types=[
            pltpu.VMEM((N_FEAT * DIM,), jnp.float32),
            pltpu.VMEM((CHUNK_ELEMS,), jnp.float32),
            pltpu.VMEM((FCHUNK,), jnp.int32),
            pltpu.VMEM((CHUNK_ELEMS,), jnp.float32),
        ],
    )
    def run(idx_hbm, w_hbm, b_hbm, out_hbm, w_v, bias_v, idx_v, out_v):
        wid = lax.axis_index("s") * nc + lax.axis_index("c")
        b0 = wid * bpw
        pltpu.sync_copy(w_hbm, w_v)
        lane = lax.iota(jnp.int32, 16)

        def fc_body(fc, _):
            pltpu.sync_copy(b_hbm.at[pl.ds(fc * CHUNK_ELEMS, CHUNK_ELEMS)], bias_v)

            def b_body(bi, _):
                brow = b0 + bi
                idx_off = pl.multiple_of(brow * N_FEAT + fc * FCHUNK, 8)
                pltpu.sync_copy(idx_hbm.at[pl.ds(idx_off, FCHUNK)], idx_v)

                def j_body(j, _):
                    jvec = jnp.full((16,), j, jnp.int32)
                    iv = plsc.load_gather(idx_v, [jvec])
                    base = iv * DIM
                    for l in range(4):
                        w = plsc.load_gather(w_v, [base + (l * 16 + lane)])
                        off = j * DIM + l * 16
                        out_v[pl.ds(off, 16)] = w + bias_v[pl.ds(off, 16)]
                    return 0

                lax.fori_loop(0, FCHUNK, j_body, 0)
                out_off = pl.multiple_of(
                    brow * (N_FEAT * DIM) + fc * CHUNK_ELEMS, 8)
                pltpu.sync_copy(out_v, out_hbm.at[pl.ds(out_off, CHUNK_ELEMS)])
                return 0

            lax.fori_loop(0, bpw, b_body, 0)
            return 0

        lax.fori_loop(0, FC, fc_body, 0)

    return run(idx, wflat, bflat).reshape(BATCH, N_FEAT * DIM)
